# baseline (speedup 1.0000x reference)
import functools

import jax
import jax.numpy as jnp
from jax import lax
from jax.experimental import pallas as pl
from jax.experimental.pallas import tpu as pltpu
from jax.experimental.pallas import tpu_sc as plsc

E = 8
H = 1024
I_MOE = 512
I_SHARED = 1024
T = 1024
NW = 32
TOK = T // NW
LANES = 16


def _dot_t(a, b):
    return jax.lax.dot_general(
        a, b, (((1,), (1,)), ((), ())), preferred_element_type=jnp.float32
    )


def _bf(v):
    return v.astype(jnp.bfloat16)


def _silu(x):
    return x * jax.nn.sigmoid(x)


def _logits_kernel(x_ref, gate_w_ref, out_ref):
    out_ref[...] = _dot_t(gate_w_ref[...], x_ref[...])


def _logits(x, gate_w):
    return pl.pallas_call(
        _logits_kernel,
        out_shape=jax.ShapeDtypeStruct((E, T), jnp.float32),
    )(x, gate_w)


def _sc_router(logits_hbm, out_hbm, lg_v, cb_v):
    wid = lax.axis_index("s") * 2 + lax.axis_index("c")
    pltpu.sync_copy(logits_hbm.at[wid], lg_v)

    for c in range(TOK // LANES):
        sl = pl.ds(c * LANES, LANES)
        lg = [lg_v[e, sl] for e in range(E)]
        m = lg[0]
        for e in range(1, E):
            m = jnp.maximum(m, lg[e])
        p = [jnp.exp(v - m) for v in lg]
        ssum = p[0]
        for e in range(1, E):
            ssum = ssum + p[e]
        p = [q / ssum for q in p]

        one = jnp.full((LANES,), 1.0, jnp.float32)
        zero = jnp.full((LANES,), 0.0, jnp.float32)

        w1 = p[0]
        for e in range(1, E):
            w1 = jnp.maximum(w1, p[e])
        taken = zero
        sel1 = []
        for e in range(E):
            eq = jnp.where(p[e] == w1, one, zero)
            s_e = eq * (one - taken)
            sel1.append(s_e)
            taken = taken + s_e

        p2 = [p[e] - sel1[e] * (p[e] + 1.0) for e in range(E)]
        w2 = p2[0]
        for e in range(1, E):
            w2 = jnp.maximum(w2, p2[e])
        taken2 = zero
        sel2 = []
        for e in range(E):
            eq = jnp.where(p2[e] == w2, one, zero)
            s_e = eq * (one - taken2)
            sel2.append(s_e)
            taken2 = taken2 + s_e

        denom = w1 + w2
        for e in range(E):
            cb_v[e, sl] = (sel1[e] * w1 + sel2[e] * w2) / denom

    pltpu.sync_copy(cb_v, out_hbm.at[wid])


def _sc_combine(logits3):
    mesh = plsc.VectorSubcoreMesh(core_axis_name="c", subcore_axis_name="s")
    k = functools.partial(
        pl.kernel,
        mesh=mesh,
        out_type=jax.ShapeDtypeStruct((NW, E, TOK), jnp.float32),
        scratch_types=[
            pltpu.VMEM((E, TOK), jnp.float32),
            pltpu.VMEM((E, TOK), jnp.float32),
        ],
    )(_sc_router)
    return k(logits3)


def _moe_kernel(x_ref, combine_ref, gp_ref, up_ref, dp_ref,
                sg_ref, su_ref, sd_ref, seg_ref,
                out_ref, xb_ref):
    s = pl.program_id(0)
    t = out_ref.shape[0]
    e_iota = jax.lax.broadcasted_iota(jnp.int32, (t, E), 1)

    @pl.when(s == 0)
    def _shared():
        x = x_ref[...]
        xb_ref[...] = _bf(x)
        xbs = xb_ref[...]
        sg = _dot_t(xbs, _bf(sg_ref[...]))
        su = _dot_t(xbs, _bf(su_ref[...]))
        sh = _dot_t(_bf(_silu(sg) * su), _bf(sd_ref[...]))
        gv = jax.nn.sigmoid(_dot_t(x, seg_ref[...]))
        out_ref[...] = gv * sh

    xb = xb_ref[...]

    g = _dot_t(xb, _bf(gp_ref[0]))
    u = _dot_t(xb, _bf(up_ref[0]))
    act = _silu(g) * u
    w_e = jnp.sum(jnp.where(e_iota == s, combine_ref[...], 0.0),
                  axis=-1, keepdims=True)
    out_ref[...] += _dot_t(_bf(act * w_e), _bf(dp_ref[0]))


def kernel(hidden_states, gate_w, gate_proj_w, up_proj_w, down_proj_w,
           shared_gate_w, shared_up_w, shared_down_w, shared_expert_gate_w):
    b, s, h = hidden_states.shape
    x = hidden_states.reshape(-1, h)
    t = x.shape[0]

    logits = _logits(x, gate_w)
    logits3 = jnp.transpose(logits.reshape(E, NW, TOK), (1, 0, 2))
    cb3 = _sc_combine(logits3)
    combine = jnp.transpose(cb3, (1, 0, 2)).reshape(E, t).T

    out = pl.pallas_call(
        _moe_kernel,
        grid=(E,),
        in_specs=[
            pl.BlockSpec((t, h), lambda i: (0, 0)),
            pl.BlockSpec((t, E), lambda i: (0, 0)),
            pl.BlockSpec((1, I_MOE, h), lambda i: (i, 0, 0)),
            pl.BlockSpec((1, I_MOE, h), lambda i: (i, 0, 0)),
            pl.BlockSpec((1, h, I_MOE), lambda i: (i, 0, 0)),
            pl.BlockSpec((I_SHARED, h), lambda i: (0, 0)),
            pl.BlockSpec((I_SHARED, h), lambda i: (0, 0)),
            pl.BlockSpec((h, I_SHARED), lambda i: (0, 0)),
            pl.BlockSpec((1, h), lambda i: (0, 0)),
        ],
        out_specs=pl.BlockSpec((t, h), lambda i: (0, 0)),
        out_shape=jax.ShapeDtypeStruct((t, h), jnp.float32),
        scratch_shapes=[
            pltpu.VMEM((t, H), jnp.bfloat16),
        ],
    )(x, combine, gate_proj_w, up_proj_w, down_proj_w,
      shared_gate_w, shared_up_w, shared_down_w, shared_expert_gate_w)

    return out.reshape(b, s, h)


# speedup vs baseline: 1.3743x; 1.3743x over previous
import jax
import jax.numpy as jnp
from jax.experimental import pallas as pl
from jax.experimental.pallas import tpu as pltpu

E = 8
H = 1024
I_MOE = 512
I_SHARED = 1024
SH_HALF = I_SHARED // 2
N_STEPS = E + 2


def _dot_t(a, b):
    return jax.lax.dot_general(
        a, b, (((1,), (1,)), ((), ())), preferred_element_type=jnp.float32
    )


def _bf(v):
    return v.astype(jnp.bfloat16)


def _silu(x):
    return x * jax.nn.sigmoid(x)


def _moe_kernel(x_ref, gate_w_ref, gp_ref, up_ref, dp_ref,
                sg_ref, su_ref, sd_ref, seg_ref,
                out_ref, combine_ref, gv_ref, xb_ref):
    s = pl.program_id(0)
    t = out_ref.shape[0]
    e_iota = jax.lax.broadcasted_iota(jnp.int32, (t, E), 1)

    @pl.when(s == 0)
    def _router():
        x = x_ref[...]
        xb_ref[...] = _bf(x)

        logits = _dot_t(x, gate_w_ref[...])
        m = jnp.max(logits, axis=-1, keepdims=True)
        p = jnp.exp(logits - m)
        p = p / jnp.sum(p, axis=-1, keepdims=True)

        w1 = jnp.max(p, axis=-1, keepdims=True)
        i1 = jnp.min(jnp.where(p == w1, e_iota, E), axis=-1, keepdims=True)
        m1 = e_iota == i1
        p2 = jnp.where(m1, -1.0, p)
        w2 = jnp.max(p2, axis=-1, keepdims=True)
        i2 = jnp.min(jnp.where(p2 == w2, e_iota, E), axis=-1, keepdims=True)
        m2 = e_iota == i2
        combine = jnp.where(m1, w1, 0.0) + jnp.where(m2, w2, 0.0)
        combine_ref[...] = combine / (w1 + w2)

        gv_ref[...] = jax.nn.sigmoid(_dot_t(x, seg_ref[...]))

    xb = xb_ref[...]
    is_shared = (s == 3) | (s == 4)

    @pl.when(~is_shared)
    def _expert():
        e_num = s - jnp.where(s >= 5, 2, 0)
        g = _dot_t(xb, _bf(gp_ref[0]))
        u = _dot_t(xb, _bf(up_ref[0]))
        act = _silu(g) * u
        w_e = jnp.sum(jnp.where(e_iota == e_num, combine_ref[...], 0.0),
                      axis=-1, keepdims=True)
        contrib = _dot_t(_bf(act * w_e), _bf(dp_ref[0]))

        @pl.when(s == 0)
        def _init():
            out_ref[...] = contrib

        @pl.when(s > 0)
        def _acc():
            out_ref[...] += contrib

    @pl.when(is_shared)
    def _shared_half():
        sg = _dot_t(xb, _bf(sg_ref[...]))
        su = _dot_t(xb, _bf(su_ref[...]))
        sh = _dot_t(_bf(_silu(sg) * su), _bf(sd_ref[...]))
        out_ref[...] += gv_ref[...] * sh


def kernel(hidden_states, gate_w, gate_proj_w, up_proj_w, down_proj_w,
           shared_gate_w, shared_up_w, shared_down_w, shared_expert_gate_w):
    b, s, h = hidden_states.shape
    x = hidden_states.reshape(-1, h)
    t = x.shape[0]

    def _ie(i):
        return jnp.where(i <= 2, i, jnp.where(i <= 4, 2, i - 2))

    def _ish(i):
        return jnp.where(i <= 3, 0, 1)

    out = pl.pallas_call(
        _moe_kernel,
        grid=(N_STEPS,),
        in_specs=[
            pl.BlockSpec((t, h), lambda i: (0, 0)),
            pl.BlockSpec((E, h), lambda i: (0, 0)),
            pl.BlockSpec((1, I_MOE, h), lambda i: (_ie(i), 0, 0)),
            pl.BlockSpec((1, I_MOE, h), lambda i: (_ie(i), 0, 0)),
            pl.BlockSpec((1, h, I_MOE), lambda i: (_ie(i), 0, 0)),
            pl.BlockSpec((SH_HALF, h), lambda i: (_ish(i), 0)),
            pl.BlockSpec((SH_HALF, h), lambda i: (_ish(i), 0)),
            pl.BlockSpec((h, SH_HALF), lambda i: (0, _ish(i))),
            pl.BlockSpec((1, h), lambda i: (0, 0)),
        ],
        out_specs=pl.BlockSpec((t, h), lambda i: (0, 0)),
        out_shape=jax.ShapeDtypeStruct((t, h), jnp.float32),
        scratch_shapes=[
            pltpu.VMEM((t, E), jnp.float32),
            pltpu.VMEM((t, 1), jnp.float32),
            pltpu.VMEM((t, H), jnp.bfloat16),
        ],
    )(x, gate_w, gate_proj_w, up_proj_w, down_proj_w,
      shared_gate_w, shared_up_w, shared_down_w, shared_expert_gate_w)

    return out.reshape(b, s, h)


# speedup vs baseline: 1.5025x; 1.0932x over previous
import jax
import jax.numpy as jnp
from jax.experimental import pallas as pl
from jax.experimental.pallas import tpu as pltpu

E = 8
H = 1024
I_MOE = 512
I_SHARED = 1024


def _dot_t(a, b):
    return jax.lax.dot_general(
        a, b, (((1,), (1,)), ((), ())), preferred_element_type=jnp.float32
    )


def _bf(v):
    return v.astype(jnp.bfloat16)


def _silu(x):
    return x * jax.nn.sigmoid(x)


def _moe_kernel(x_ref, gate_w_ref, gp_ref, up_ref, dp_ref,
                sg_ref, su_ref, sd_ref, seg_ref,
                out_ref, combine_ref, xb_ref):
    s = pl.program_id(0)
    t = out_ref.shape[0]
    e_iota = jax.lax.broadcasted_iota(jnp.int32, (t, E), 1)

    @pl.when(s == 0)
    def _router_and_shared():
        x = x_ref[...]
        xb_ref[...] = _bf(x)

        logits = _dot_t(x, gate_w_ref[...])
        m = jnp.max(logits, axis=-1, keepdims=True)
        p = jnp.exp(logits - m)
        p = p / jnp.sum(p, axis=-1, keepdims=True)

        w1 = jnp.max(p, axis=-1, keepdims=True)
        i1 = jnp.min(jnp.where(p == w1, e_iota, E), axis=-1, keepdims=True)
        m1 = e_iota == i1
        p2 = jnp.where(m1, -1.0, p)
        w2 = jnp.max(p2, axis=-1, keepdims=True)
        i2 = jnp.min(jnp.where(p2 == w2, e_iota, E), axis=-1, keepdims=True)
        m2 = e_iota == i2
        combine = jnp.where(m1, w1, 0.0) + jnp.where(m2, w2, 0.0)
        combine_ref[...] = combine / (w1 + w2)

        xbs = xb_ref[...]
        sg = _dot_t(xbs, _bf(sg_ref[...]))
        su = _dot_t(xbs, _bf(su_ref[...]))
        sh = _dot_t(_bf(_silu(sg) * su), _bf(sd_ref[...]))
        gv = jax.nn.sigmoid(_dot_t(x, seg_ref[...]))
        out_ref[...] = gv * sh

    xb = xb_ref[...]

    g = _dot_t(xb, _bf(gp_ref[0]))
    u = _dot_t(xb, _bf(up_ref[0]))
    act = _silu(g) * u
    w_e = jnp.sum(jnp.where(e_iota == s, combine_ref[...], 0.0),
                  axis=-1, keepdims=True)
    out_ref[...] += _dot_t(_bf(act * w_e), _bf(dp_ref[0]))


def kernel(hidden_states, gate_w, gate_proj_w, up_proj_w, down_proj_w,
           shared_gate_w, shared_up_w, shared_down_w, shared_expert_gate_w):
    b, s, h = hidden_states.shape
    x = hidden_states.reshape(-1, h)
    t = x.shape[0]

    out = pl.pallas_call(
        _moe_kernel,
        grid=(E,),
        in_specs=[
            pl.BlockSpec((t, h), lambda i: (0, 0)),
            pl.BlockSpec((E, h), lambda i: (0, 0)),
            pl.BlockSpec((1, I_MOE, h), lambda i: (i, 0, 0)),
            pl.BlockSpec((1, I_MOE, h), lambda i: (i, 0, 0)),
            pl.BlockSpec((1, h, I_MOE), lambda i: (i, 0, 0)),
            pl.BlockSpec((I_SHARED, h), lambda i: (0, 0)),
            pl.BlockSpec((I_SHARED, h), lambda i: (0, 0)),
            pl.BlockSpec((h, I_SHARED), lambda i: (0, 0)),
            pl.BlockSpec((1, h), lambda i: (0, 0)),
        ],
        out_specs=pl.BlockSpec((t, h), lambda i: (0, 0)),
        out_shape=jax.ShapeDtypeStruct((t, h), jnp.float32),
        scratch_shapes=[
            pltpu.VMEM((t, E), jnp.float32),
            pltpu.VMEM((t, H), jnp.bfloat16),
        ],
    )(x, gate_w, gate_proj_w, up_proj_w, down_proj_w,
      shared_gate_w, shared_up_w, shared_down_w, shared_expert_gate_w)

    return out.reshape(b, s, h)
